# Initial kernel scaffold; baseline (speedup 1.0000x reference)
import jax
import jax.numpy as jnp
from jax.experimental import pallas as pl


def kernel(x, x_p, edge_index):
    raise NotImplementedError("write your pallas kernel here")


# speedup vs baseline: 5.9405x; 5.9405x over previous
import functools

import jax
import jax.numpy as jnp
from jax import lax
from jax.experimental import pallas as pl
from jax.experimental.pallas import tpu as pltpu
from jax.experimental.pallas import tpu_sc as plsc

C1 = 1e-6
C2 = 1e-6
NW = 32
G = 4


def _stats_body(x_ref, m_ref, v_ref):
    xb = x_ref[...]
    m = jnp.mean(xb, axis=0, keepdims=True)
    q = jnp.mean(xb * xb, axis=0, keepdims=True)
    m_ref[...] = m
    v_ref[...] = q - m * m


def _make_sc_kernel(n_pad, c, k, npw, ng):
    ge = G * k
    cc = c // 16
    mesh = plsc.VectorSubcoreMesh(core_axis_name="c", subcore_axis_name="s",
                                  num_cores=2, num_subcores=16)

    @functools.partial(
        pl.kernel,
        out_type=jax.ShapeDtypeStruct((n_pad, c), jnp.float32),
        mesh=mesh,
        compiler_params=pltpu.CompilerParams(needs_layout_passes=False),
        scratch_types=[
            pltpu.VMEM((n_pad,), jnp.float32),
            pltpu.VMEM((n_pad,), jnp.float32),
            pltpu.VMEM((ng, ge), jnp.int32),
            pltpu.VMEM((ng, ge), jnp.int32),
            pltpu.VMEM((ge, 2 * c), jnp.float32),
            pltpu.VMEM((ge, 2 * c), jnp.float32),
            pltpu.VMEM((ge, 2 * c), jnp.float32),
            pltpu.VMEM((ge, 2 * c), jnp.float32),
            pltpu.VMEM((G, c), jnp.float32),
            pltpu.VMEM((16,), jnp.float32),
            pltpu.SemaphoreType.DMA,
            pltpu.SemaphoreType.DMA,
            pltpu.SemaphoreType.DMA,
            pltpu.SemaphoreType.DMA,
        ],
    )
    def sc_kernel(xall_h, eii_h, eij_h, m_h, v_h, out_h,
                  m_v, v_v, eii_v, eij_v,
                  ibuf0, ibuf1, jbuf0, jbuf1, obuf, sffb,
                  si0, si1, sj0, sj1):
        wid = lax.axis_index("s") * 2 + lax.axis_index("c")
        node0 = wid * npw

        pltpu.sync_copy(m_h, m_v)
        pltpu.sync_copy(v_h, v_v)
        pltpu.sync_copy(eii_h.at[wid], eii_v)
        pltpu.sync_copy(eij_h.at[wid], eij_v)

        def start(g, ib, jb, si, sj):
            pltpu.async_copy(xall_h.at[eii_v.at[g]], ib, si)
            pltpu.async_copy(xall_h.at[eij_v.at[g]], jb, sj)

        def wait(g, ib, jb, si, sj):
            pltpu.make_async_copy(xall_h.at[eii_v.at[g]], ib, si).wait()
            pltpu.make_async_copy(xall_h.at[eij_v.at[g]], jb, sj).wait()

        lane = lax.iota(jnp.int32, 16)

        def process(g, ib, jb):
            for t in range(G):
                row0 = t * k
                iidx = eii_v[g, pl.ds(row0, 16)]
                jidx = eij_v[g, pl.ds(row0, 16)]
                mi = plsc.load_gather(m_v, [iidx])
                mj = plsc.load_gather(m_v, [jidx])
                vi = plsc.load_gather(v_v, [iidx])
                vj = plsc.load_gather(v_v, [jidx])
                rows = row0 + lane

                def dot_body(ci, acc):
                    for u in range(8):
                        col = jnp.full((16,), ci * 8 + u, jnp.int32)
                        a = plsc.load_gather(ib, [rows, col])
                        b = plsc.load_gather(jb, [rows, col])
                        acc = acc + a * b
                    return acc

                dot = lax.fori_loop(0, c // 8, dot_body,
                                    jnp.zeros((16,), jnp.float32))
                cov = dot * (1.0 / c) - mi * mj
                s1 = (2.0 * mi * mj + C1) / (mi * mi + mj * mj + C1)
                s2 = (2.0 * cov + C2) / (vi + vj + C2)
                sffb[...] = 1.0 - s1 * s2

                def k_body(e, accs):
                    s = plsc.load_gather(sffb, [jnp.full((16,), e, jnp.int32)])
                    out = []
                    for q in range(cc):
                        a = ib[row0 + e, pl.ds(c + q * 16, 16)]
                        b = jb[row0 + e, pl.ds(c + q * 16, 16)]
                        out.append(accs[q] + a + b + jnp.abs(a - b) * s)
                    return tuple(out)

                accs = lax.fori_loop(
                    0, k, k_body,
                    tuple(jnp.zeros((16,), jnp.float32) for _ in range(cc)))
                for q in range(cc):
                    obuf[t, pl.ds(q * 16, 16)] = accs[q]
            pltpu.sync_copy(obuf, out_h.at[pl.ds(node0 + g * G, G)])

        start(0, ibuf0, jbuf0, si0, sj0)

        def outer(tt, carry):
            g0 = 2 * tt
            start(g0 + 1, ibuf1, jbuf1, si1, sj1)
            wait(g0, ibuf0, jbuf0, si0, sj0)
            process(g0, ibuf0, jbuf0)
            g1 = g0 + 1

            @pl.when(g1 + 1 < ng)
            def _():
                start(g1 + 1, ibuf0, jbuf0, si0, sj0)

            wait(g1, ibuf1, jbuf1, si1, sj1)
            process(g1, ibuf1, jbuf1)
            return carry

        lax.fori_loop(0, ng // 2, outer, 0)

    return sc_kernel


def kernel(x, x_p, edge_index):
    b, c, n, _ = x.shape
    k = edge_index.shape[3]
    npw = -(-n // (NW * 2 * G)) * 2 * G
    n_pad = NW * npw
    ng = npw // G

    x2 = x[0, :, :, 0]
    xp2 = x_p[0, :, :, 0]
    pad = ((0, 0), (0, n_pad - n))
    x2p = jnp.pad(x2, pad)
    xp2p = jnp.pad(xp2, pad)

    m2, v2 = pl.pallas_call(
        _stats_body,
        out_shape=(jax.ShapeDtypeStruct((1, n_pad), jnp.float32),
                   jax.ShapeDtypeStruct((1, n_pad), jnp.float32)),
    )(x2p)

    xall = jnp.concatenate([x2p.T, xp2p.T], axis=1)
    ei = jnp.pad(edge_index[:, 0], ((0, 0), (0, n_pad - n), (0, 0)))
    eii = ei[1].reshape(NW, ng, G * k)
    eij = ei[0].reshape(NW, ng, G * k)

    sc = _make_sc_kernel(n_pad, c, k, npw, ng)
    out = sc(xall, eii, eij, m2.reshape(n_pad), v2.reshape(n_pad))
    return out[:n].T[None, :, :, None]


# speedup vs baseline: 7.7661x; 1.3073x over previous
import functools

import jax
import jax.numpy as jnp
from jax import lax
from jax.experimental import pallas as pl
from jax.experimental.pallas import tpu as pltpu
from jax.experimental.pallas import tpu_sc as plsc

C1 = 1e-6
C2 = 1e-6
NW = 32
G = 4


def _stats_body(x_ref, m_ref, v_ref):
    xb = x_ref[...]
    m = jnp.mean(xb, axis=0, keepdims=True)
    q = jnp.mean(xb * xb, axis=0, keepdims=True)
    m_ref[...] = m
    v_ref[...] = q - m * m


def _make_sc_kernel(n_pad, c, k, npw, ng):
    ge = G * k
    cc = c // 16
    mesh = plsc.VectorSubcoreMesh(core_axis_name="c", subcore_axis_name="s",
                                  num_cores=2, num_subcores=16)

    @functools.partial(
        pl.kernel,
        out_type=jax.ShapeDtypeStruct((n_pad, c), jnp.float32),
        mesh=mesh,
        compiler_params=pltpu.CompilerParams(needs_layout_passes=False),
        scratch_types=[
            pltpu.VMEM((n_pad,), jnp.float32),
            pltpu.VMEM((n_pad,), jnp.float32),
            pltpu.VMEM((ng, ge), jnp.int32),
            pltpu.VMEM((ng, ge), jnp.int32),
            pltpu.VMEM((ge, 2 * c), jnp.float32),
            pltpu.VMEM((ge, 2 * c), jnp.float32),
            pltpu.VMEM((ge, 2 * c), jnp.float32),
            pltpu.VMEM((ge, 2 * c), jnp.float32),
            pltpu.VMEM((G, c), jnp.float32),
            pltpu.VMEM((16,), jnp.float32),
            pltpu.SemaphoreType.DMA,
            pltpu.SemaphoreType.DMA,
            pltpu.SemaphoreType.DMA,
            pltpu.SemaphoreType.DMA,
        ],
    )
    def sc_kernel(xall_h, eii_h, eij_h, m_h, v_h, out_h,
                  m_v, v_v, eii_v, eij_v,
                  ibuf0, ibuf1, jbuf0, jbuf1, obuf, sffb,
                  si0, si1, sj0, sj1):
        wid = lax.axis_index("s") * 2 + lax.axis_index("c")
        node0 = wid * npw

        pltpu.sync_copy(m_h, m_v)
        pltpu.sync_copy(v_h, v_v)
        pltpu.sync_copy(eii_h.at[wid], eii_v)
        pltpu.sync_copy(eij_h.at[wid], eij_v)

        def start(g, ib, jb, si, sj):
            pltpu.async_copy(xall_h.at[eii_v.at[g]], ib, si)
            pltpu.async_copy(xall_h.at[eij_v.at[g]], jb, sj)

        def wait(g, ib, jb, si, sj):
            pltpu.make_async_copy(xall_h.at[eii_v.at[g]], ib, si).wait()
            pltpu.make_async_copy(xall_h.at[eij_v.at[g]], jb, sj).wait()

        def process(g, ib, jb):
            def node_body(t, carry):
                row0 = t * k
                iidx = eii_v[g, pl.ds(row0, 16)]
                jidx = eij_v[g, pl.ds(row0, 16)]
                mi = plsc.load_gather(m_v, [iidx])
                mj = plsc.load_gather(m_v, [jidx])
                vi = plsc.load_gather(v_v, [iidx])
                vj = plsc.load_gather(v_v, [jidx])
                mmv = mi * mj
                s1v = (2.0 * mmv + C1) / (mi * mi + mj * mj + C1)
                s1dv = s1v / (vi + vj + C2)
                accs = [jnp.zeros((16,), jnp.float32) for _ in range(cc)]
                for e in range(k):
                    av = [ib[row0 + e, pl.ds(q * 16, 16)] for q in range(cc)]
                    bv = [jb[row0 + e, pl.ds(q * 16, 16)] for q in range(cc)]
                    d0 = av[0] * bv[0]
                    d1 = av[1] * bv[1]
                    for q in range(2, cc, 2):
                        d0 = d0 + av[q] * bv[q]
                        d1 = d1 + av[q + 1] * bv[q + 1]
                    dot = jnp.sum(d0 + d1)
                    cov2 = 2.0 * (dot * (1.0 / c) - mmv[e]) + C2
                    sff = jnp.full((16,), 1.0 - s1dv[e] * cov2, jnp.float32)
                    for q in range(cc):
                        a = ib[row0 + e, pl.ds(c + q * 16, 16)]
                        b = jb[row0 + e, pl.ds(c + q * 16, 16)]
                        accs[q] = accs[q] + (a + b) + jnp.abs(a - b) * sff
                for q in range(cc):
                    obuf[t, pl.ds(q * 16, 16)] = accs[q]
                return carry

            lax.fori_loop(0, G, node_body, 0)
            pltpu.sync_copy(obuf, out_h.at[pl.ds(node0 + g * G, G)])

        start(0, ibuf0, jbuf0, si0, sj0)

        def outer(tt, carry):
            g0 = 2 * tt
            start(g0 + 1, ibuf1, jbuf1, si1, sj1)
            wait(g0, ibuf0, jbuf0, si0, sj0)
            process(g0, ibuf0, jbuf0)
            g1 = g0 + 1

            @pl.when(g1 + 1 < ng)
            def _():
                start(g1 + 1, ibuf0, jbuf0, si0, sj0)

            wait(g1, ibuf1, jbuf1, si1, sj1)
            process(g1, ibuf1, jbuf1)
            return carry

        lax.fori_loop(0, ng // 2, outer, 0)

    return sc_kernel


def kernel(x, x_p, edge_index):
    b, c, n, _ = x.shape
    k = edge_index.shape[3]
    npw = -(-n // (NW * 2 * G)) * 2 * G
    n_pad = NW * npw
    ng = npw // G

    x2 = x[0, :, :, 0]
    xp2 = x_p[0, :, :, 0]
    pad = ((0, 0), (0, n_pad - n))
    x2p = jnp.pad(x2, pad)
    xp2p = jnp.pad(xp2, pad)

    m2, v2 = pl.pallas_call(
        _stats_body,
        out_shape=(jax.ShapeDtypeStruct((1, n_pad), jnp.float32),
                   jax.ShapeDtypeStruct((1, n_pad), jnp.float32)),
    )(x2p)

    xall = jnp.concatenate([x2p.T, xp2p.T], axis=1)
    ei = jnp.pad(edge_index[:, 0], ((0, 0), (0, n_pad - n), (0, 0)))
    eii = ei[1].reshape(NW, ng, G * k)
    eij = ei[0].reshape(NW, ng, G * k)

    sc = _make_sc_kernel(n_pad, c, k, npw, ng)
    out = sc(xall, eii, eij, m2.reshape(n_pad), v2.reshape(n_pad))
    return out[:n].T[None, :, :, None]
